# baseline (speedup 1.0000x reference)
import functools

import jax
import jax.numpy as jnp
from jax import lax
from jax.experimental import pallas as pl
from jax.experimental.pallas import tpu as pltpu
from jax.experimental.pallas import tpu_sc as plsc

_NC = 2
_NS = 16
_NW = _NC * _NS

_B = 4096
_EMB = 64
_NAME_LEN = 20
_DESC_LEN = 200
_DHALF = 100
_FEAT = 2 * _EMB
_BPW = _B // _NW

_LANES = 16
_EC = _EMB // _LANES


def _pool_sc_kernel(name_hbm, desc_hbm, emb_hbm, out_hbm,
                    nidx, didx, nrows, drows0, drows1, stage,
                    sem_n, sem_d0, sem_d1):
  wid = lax.axis_index("s") * _NC + lax.axis_index("c")
  base = wid * _BPW

  pltpu.sync_copy(name_hbm.at[pl.ds(base, _BPW)], nidx)
  pltpu.sync_copy(desc_hbm.at[pl.ds(2 * base, 2 * _BPW)], didx)

  @pl.loop(0, _BPW)
  def _row(r):
    cp_n = pltpu.async_copy(emb_hbm.at[nidx.at[r]], nrows, sem_n)
    cp_d0 = pltpu.async_copy(emb_hbm.at[didx.at[2 * r]], drows0, sem_d0)
    cp_d1 = pltpu.async_copy(emb_hbm.at[didx.at[2 * r + 1]], drows1, sem_d1)
    cp_n.wait()

    zeros = jnp.zeros((_LANES,), jnp.float32)

    def nbody(j, acc):
      return tuple(acc[k] + nrows[j, pl.ds(k * _LANES, _LANES)]
                   for k in range(_EC))
    nacc = lax.fori_loop(0, _NAME_LEN, nbody, (zeros,) * _EC)

    cp_d0.wait()

    def d0body(j, acc):
      return tuple(acc[k] + drows0[j, pl.ds(k * _LANES, _LANES)]
                   for k in range(_EC))
    dacc = lax.fori_loop(0, _DHALF, d0body, (zeros,) * _EC)

    cp_d1.wait()

    def d1body(j, acc):
      return tuple(acc[k] + drows1[j, pl.ds(k * _LANES, _LANES)]
                   for k in range(_EC))
    dacc = lax.fori_loop(0, _DHALF, d1body, dacc)

    for k in range(_EC):
      stage[r, pl.ds(k * _LANES, _LANES)] = nacc[k]
      stage[r, pl.ds(_EMB + k * _LANES, _LANES)] = dacc[k]

  pltpu.sync_copy(stage, out_hbm.at[pl.ds(base, _BPW)])


@jax.jit
def _pool(name, desc2, emb):
  mesh = plsc.VectorSubcoreMesh(core_axis_name="c", subcore_axis_name="s")
  f = pl.kernel(
      _pool_sc_kernel,
      out_type=jax.ShapeDtypeStruct((_B, _FEAT), jnp.float32),
      mesh=mesh,
      compiler_params=pltpu.CompilerParams(use_tc_tiling_on_sc=False),
      scratch_types=[
          pltpu.VMEM((_BPW, _NAME_LEN), jnp.int32),
          pltpu.VMEM((2 * _BPW, _DHALF), jnp.int32),
          pltpu.VMEM((_NAME_LEN, _EMB), jnp.float32),
          pltpu.VMEM((_DHALF, _EMB), jnp.float32),
          pltpu.VMEM((_DHALF, _EMB), jnp.float32),
          pltpu.VMEM((_BPW, _FEAT), jnp.float32),
          pltpu.SemaphoreType.DMA,
          pltpu.SemaphoreType.DMA,
          pltpu.SemaphoreType.DMA,
      ],
  )
  return f(name, desc2, emb)


def _mlp_tc_kernel(feat_ref, nlen_ref, dlen_ref, w1t_ref, b1_ref, w2t_ref,
                   b2_ref, out_ref):
  feat = feat_ref[...]
  n = feat[:, :_EMB] / nlen_ref[...]
  d = feat[:, _EMB:] / dlen_ref[...]
  f = jnp.concatenate([n, d], axis=1)
  y1 = jnp.dot(f, w1t_ref[...], preferred_element_type=jnp.float32)
  y1 = y1 + b1_ref[...]
  y = jnp.dot(y1, w2t_ref[...], preferred_element_type=jnp.float32)
  out_ref[...] = y + b2_ref[...]


@jax.jit
def _mlp(feat, nlen, dlen, w1t, b1, w2t, b2):
  bt = 512
  grid = (_B // bt,)
  hid = w1t.shape[1]
  lab = w2t.shape[1]
  return pl.pallas_call(
      _mlp_tc_kernel,
      grid=grid,
      in_specs=[
          pl.BlockSpec((bt, _FEAT), lambda i: (i, 0)),
          pl.BlockSpec((bt, 1), lambda i: (i, 0)),
          pl.BlockSpec((bt, 1), lambda i: (i, 0)),
          pl.BlockSpec((_FEAT, hid), lambda i: (0, 0)),
          pl.BlockSpec((1, hid), lambda i: (0, 0)),
          pl.BlockSpec((hid, lab), lambda i: (0, 0)),
          pl.BlockSpec((1, lab), lambda i: (0, 0)),
      ],
      out_specs=pl.BlockSpec((bt, lab), lambda i: (i, 0)),
      out_shape=jax.ShapeDtypeStruct((_B, lab), jnp.float32),
  )(feat, nlen, dlen, w1t, b1, w2t, b2)


def kernel(name, name_length, name_mask, desc, desc_length, desc_mask,
           emb, W1, b1, W2, b2):
  desc2 = desc.reshape(2 * _B, _DHALF)
  feat = _pool(name, desc2, emb)
  nlen = name_length.astype(jnp.float32).reshape(_B, 1)
  dlen = desc_length.astype(jnp.float32).reshape(_B, 1)
  return _mlp(feat, nlen, dlen, W1.T, b1.reshape(1, -1), W2.T,
              b2.reshape(1, -1))
